# baseline (speedup 1.0000x reference)
import functools

import jax
import jax.numpy as jnp
from jax import lax
from jax.experimental import pallas as pl
from jax.experimental.pallas import tpu as pltpu
from jax.experimental.pallas import tpu_sc as plsc

NTAGS = 16
NTOK = 16384
NC = 2
NS = 16
NW = NC * NS
BPW = NTOK // NW
CHUNK = 128
NCHUNK = BPW // CHUNK

_mesh = plsc.VectorSubcoreMesh(core_axis_name="c", subcore_axis_name="s")


@functools.partial(
    pl.kernel,
    out_type=jax.ShapeDtypeStruct((NC, NTAGS), jnp.float32),
    mesh=_mesh,
    scratch_types=[
        pltpu.VMEM((NCHUNK, CHUNK), jnp.int32),
        pltpu.VMEM((NCHUNK, CHUNK, NTAGS), jnp.float32),
        pltpu.VMEM((NTAGS,), jnp.float32),
        pltpu.VMEM((NS, NTAGS), jnp.float32),
        pltpu.VMEM((NTAGS,), jnp.float32),
        pltpu.VMEM_SHARED((NS, NTAGS), jnp.float32),
        pltpu.SemaphoreType.DMA,
    ],
    compiler_params=pltpu.CompilerParams(use_tc_tiling_on_sc=False),
)
def _bow_sc(x_hbm, table_hbm, bias_hbm, out_hbm,
            idx_v, rows_v, acc_v, comb_v, bias_v, sh, sem):
    cid = lax.axis_index("c")
    sid = lax.axis_index("s")
    wid = sid * NC + cid

    pltpu.sync_copy(x_hbm.at[wid], idx_v)

    copies = [
        pltpu.async_copy(table_hbm.at[idx_v.at[j]], rows_v.at[j], sem)
        for j in range(NCHUNK)
    ]
    for c in copies:
        c.wait()

    def body(i, accs):
        return tuple(accs[j] + rows_v[j, i, :] for j in range(NCHUNK))

    accs = lax.fori_loop(
        0, CHUNK, body,
        tuple(jnp.zeros((NTAGS,), jnp.float32) for _ in range(NCHUNK)),
    )
    total = (accs[0] + accs[1]) + (accs[2] + accs[3])
    acc_v[...] = total

    pltpu.sync_copy(acc_v, sh.at[sid])
    plsc.subcore_barrier()

    @pl.when(sid == 0)
    def _():
        pltpu.sync_copy(sh, comb_v)
        pltpu.sync_copy(bias_hbm, bias_v)
        core_sum = comb_v[0, :]
        for t in range(1, NS):
            core_sum = core_sum + comb_v[t, :]

        @pl.when(cid == 0)
        def _():
            acc_v[...] = core_sum + bias_v[...]

        @pl.when(cid != 0)
        def _():
            acc_v[...] = core_sum

        pltpu.sync_copy(acc_v, out_hbm.at[cid])


def kernel(x, table, bias):
    x4 = x.reshape(NW, NCHUNK, CHUNK)
    partials = _bow_sc(x4, table, bias)
    return (partials[0] + partials[1]).reshape(1, NTAGS)
